# baseline (speedup 1.0000x reference)
import functools

import jax
import jax.numpy as jnp
from jax import lax
from jax.experimental import pallas as pl
from jax.experimental.pallas import tpu as pltpu
from jax.experimental.pallas import tpu_sc as plsc

TOKENS = 32768
D_IN = 768
D_HID = 384
N_EXPERTS = 64
LB_WEIGHT = 0.01

BT = 2048
NSPLIT = 4
SEG = TOKENS // NSPLIT

NC = 2
NS = 16
NW = NC * NS
TPW = SEG // NW
CHUNK = 128
NCH = TPW // CHUNK
NG = CHUNK // 16
ACC = NS * N_EXPERTS

NCHAIN = 4
EPC = N_EXPERTS // NCHAIN


def _mlp_kernel(xa_ref, xb_ref, W1a_ref, W1b_ref, b1_ref, g_ref, be_ref,
                W2_ref, b2_ref, out_ref):
    h = jnp.dot(xa_ref[...], W1a_ref[...],
                preferred_element_type=jnp.float32)
    h = h + jnp.dot(xb_ref[...], W1b_ref[...],
                    preferred_element_type=jnp.float32)
    h = h + b1_ref[...]
    mu = jnp.mean(h, axis=-1, keepdims=True)
    c = h - mu
    var = jnp.mean(c * c, axis=-1, keepdims=True)
    hn = c / jnp.sqrt(var + 1e-5) * g_ref[...] + be_ref[...]
    hg = 0.5 * hn * (1.0 + jax.lax.erf(hn * 0.7071067811865476))
    logits_t = lax.dot_general(W2_ref[...], hg, (((0,), (1,)), ((), ())),
                               preferred_element_type=jnp.float32)
    out_ref[...] = logits_t + b2_ref[...]


def _merge2(a, b):
    am1, ai1, am2, ai2 = a
    bm1, bi1, bm2, bi2 = b
    c = bm1 > am1
    m1 = jnp.where(c, bm1, am1)
    i1 = jnp.where(c, bi1, ai1)
    cand = jnp.where(c, am1, bm1)
    candi = jnp.where(c, ai1, bi1)
    om2 = jnp.where(c, bm2, am2)
    omi = jnp.where(c, bi2, ai2)
    c2 = cand >= om2
    m2 = jnp.where(c2, cand, om2)
    i2 = jnp.where(c2, candi, omi)
    return (m1, i1, m2, i2)


def _route_kernel(lg_hbm, w_hbm, f_hbm, p_hbm, lbuf, wbuf, facc, pacc,
                  lsem, wsem):
    wid = lax.axis_index("s") * NC + lax.axis_index("c")
    lane = lax.iota(jnp.int32, 16)
    lane64 = lane * N_EXPERTS
    zeros16 = jnp.zeros((16,), jnp.float32)
    ones16 = jnp.ones((16,), jnp.float32)

    def zacc(k, _):
        facc[pl.ds(k * 16, 16)] = zeros16
        pacc[pl.ds(k * 16, 16)] = zeros16
        return 0

    lax.fori_loop(0, ACC // 16, zacc, 0)

    base_tok = wid * TPW

    def start_in(c):
        return pltpu.async_copy(
            lg_hbm.at[:, pl.ds(base_tok + c * CHUNK, CHUNK)],
            lbuf.at[c % 2], lsem)

    in_cp = {0: start_in(0)}
    out_cp = {}
    for c in range(NCH):
        if c + 1 < NCH:
            in_cp[c + 1] = start_in(c + 1)
        if c >= 2:
            out_cp[c - 2].wait()
        lb = lbuf.at[c % 2]
        wb = wbuf.at[c % 2]

        def zw(k, _):
            for j in range(4):
                wb[k, pl.ds(j * 16, 16)] = zeros16
            return 0

        lax.fori_loop(0, CHUNK, zw, 0)
        in_cp[c].wait()

        def group(g, _):
            rows = lane + g * 16
            states = []
            for ch in range(NCHAIN):
                states.append((jnp.full((16,), -jnp.inf, jnp.float32),
                               jnp.zeros((16,), jnp.int32),
                               jnp.full((16,), -jnp.inf, jnp.float32),
                               jnp.zeros((16,), jnp.int32)))
            for e in range(EPC):
                for ch in range(NCHAIN):
                    ee = ch * EPC + e
                    v = lb[ee, pl.ds(g * 16, 16)]
                    m1, i1, m2, i2 = states[ch]
                    ev = jnp.full((16,), ee, jnp.int32)
                    c1 = v > m1
                    c2 = v > m2
                    m2 = jnp.where(c1, m1, jnp.where(c2, v, m2))
                    i2 = jnp.where(c1, i1, jnp.where(c2, ev, i2))
                    m1 = jnp.where(c1, v, m1)
                    i1 = jnp.where(c1, ev, i1)
                    states[ch] = (m1, i1, m2, i2)
            m1, i1, m2, i2 = _merge2(_merge2(states[0], states[1]),
                                     _merge2(states[2], states[3]))
            e2 = jnp.exp(m2 - m1)
            w1 = 1.0 / (1.0 + e2)
            w2 = e2 * w1
            plsc.store_scatter(wb, [rows, i1], w1)
            plsc.store_scatter(wb, [rows, i2], w2)
            f2 = (w2 > 0).astype(jnp.float32)
            plsc.addupdate_scatter(facc, [lane64 + i1], ones16)
            plsc.addupdate_scatter(facc, [lane64 + i2], f2)
            plsc.addupdate_scatter(pacc, [lane64 + i1], w1)
            plsc.addupdate_scatter(pacc, [lane64 + i2], w2)
            return 0

        lax.fori_loop(0, NG, group, 0)
        out_cp[c] = pltpu.async_copy(
            wb, w_hbm.at[pl.ds(base_tok + c * CHUNK, CHUNK)], wsem)

    out_cp[NCH - 2].wait()
    out_cp[NCH - 1].wait()
    pltpu.sync_copy(facc, f_hbm.at[wid])
    pltpu.sync_copy(pacc, p_hbm.at[wid])


def _concat_kernel(a_ref, b_ref, c_ref, d_ref, out_ref):
    out_ref[pl.ds(0 * SEG, SEG)] = a_ref[...]
    out_ref[pl.ds(1 * SEG, SEG)] = b_ref[...]
    out_ref[pl.ds(2 * SEG, SEG)] = c_ref[...]
    out_ref[pl.ds(3 * SEG, SEG)] = d_ref[...]


def _lb_kernel(f_ref, p_ref, lb_ref):
    f = jnp.sum(f_ref[...], axis=0, keepdims=True)
    p = jnp.sum(p_ref[...], axis=0, keepdims=True)
    s = jnp.sum(f * p)
    lb_ref[...] = (LB_WEIGHT * N_EXPERTS / (TOKENS * TOKENS) * s
                   ).reshape(1, 1)


def _make_mlp(seg_idx):
    nblk = SEG // BT
    return pl.pallas_call(
        _mlp_kernel,
        grid=(nblk,),
        in_specs=[
            pl.BlockSpec((BT, D_IN // 2),
                         lambda i, s=seg_idx, n=nblk: (s * n + i, 0)),
            pl.BlockSpec((BT, D_IN // 2),
                         lambda i, s=seg_idx, n=nblk: (s * n + i, 1)),
            pl.BlockSpec((D_IN // 2, D_HID), lambda i: (0, 0)),
            pl.BlockSpec((D_IN // 2, D_HID), lambda i: (1, 0)),
            pl.BlockSpec((1, D_HID), lambda i: (0, 0)),
            pl.BlockSpec((1, D_HID), lambda i: (0, 0)),
            pl.BlockSpec((1, D_HID), lambda i: (0, 0)),
            pl.BlockSpec((D_HID, N_EXPERTS), lambda i: (0, 0)),
            pl.BlockSpec((N_EXPERTS, 1), lambda i: (0, 0)),
        ],
        out_specs=pl.BlockSpec((N_EXPERTS, BT), lambda i: (0, i)),
        out_shape=jax.ShapeDtypeStruct((N_EXPERTS, SEG), jnp.float32),
        compiler_params=pltpu.CompilerParams(
            dimension_semantics=("arbitrary",),
        ),
    )


_route = pl.kernel(
    _route_kernel,
    out_type=[
        jax.ShapeDtypeStruct((SEG, N_EXPERTS), jnp.float32),
        jax.ShapeDtypeStruct((NW, ACC), jnp.float32),
        jax.ShapeDtypeStruct((NW, ACC), jnp.float32),
    ],
    mesh=plsc.VectorSubcoreMesh(core_axis_name="c", subcore_axis_name="s"),
    scratch_types=[
        pltpu.VMEM((2, N_EXPERTS, CHUNK), jnp.float32),
        pltpu.VMEM((2, CHUNK, N_EXPERTS), jnp.float32),
        pltpu.VMEM((ACC,), jnp.float32),
        pltpu.VMEM((ACC,), jnp.float32),
        pltpu.SemaphoreType.DMA,
        pltpu.SemaphoreType.DMA,
    ],
    compiler_params=pltpu.CompilerParams(needs_layout_passes=False),
)


@functools.partial(jax.jit)
def kernel(x, W1, b1, gamma, beta, W2, b2):
    b1r = b1.reshape(1, D_HID)
    gr = gamma.reshape(1, D_HID)
    ber = beta.reshape(1, D_HID)
    b2r = b2.reshape(N_EXPERTS, 1)

    w_segs, f_segs, p_segs = [], [], []
    for s in range(NSPLIT):
        logits = _make_mlp(s)(x, x, W1, W1, b1r, gr, ber, W2, b2r)
        w_s, f_s, p_s = _route(logits)
        w_segs.append(w_s)
        f_segs.append(f_s)
        p_segs.append(p_s)

    weights = pl.pallas_call(
        _concat_kernel,
        in_specs=[pl.BlockSpec((SEG, N_EXPERTS), lambda: (0, 0))
                  for _ in range(NSPLIT)],
        out_specs=pl.BlockSpec((TOKENS, N_EXPERTS), lambda: (0, 0)),
        out_shape=jax.ShapeDtypeStruct((TOKENS, N_EXPERTS), jnp.float32),
    )(*w_segs)
    fpart = jnp.concatenate(f_segs, axis=0).reshape(NSPLIT * NW * NS,
                                                    N_EXPERTS)
    ppart = jnp.concatenate(p_segs, axis=0).reshape(NSPLIT * NW * NS,
                                                    N_EXPERTS)

    lb = pl.pallas_call(
        _lb_kernel,
        in_specs=[
            pl.BlockSpec((NSPLIT * NW * NS, N_EXPERTS), lambda: (0, 0)),
            pl.BlockSpec((NSPLIT * NW * NS, N_EXPERTS), lambda: (0, 0)),
        ],
        out_specs=pl.BlockSpec((1, 1), lambda: (0, 0)),
        out_shape=jax.ShapeDtypeStruct((1, 1), jnp.float32),
    )(fpart, ppart)
    return weights, lb.reshape(())


# speedup vs baseline: 1.0952x; 1.0952x over previous
import functools

import jax
import jax.numpy as jnp
from jax import lax
from jax.experimental import pallas as pl
from jax.experimental.pallas import tpu as pltpu
from jax.experimental.pallas import tpu_sc as plsc

TOKENS = 32768
D_IN = 768
D_HID = 384
N_EXPERTS = 64
LB_WEIGHT = 0.01

BT = 2048
NSPLIT = 2
SEG = TOKENS // NSPLIT

NC = 2
NS = 16
NW = NC * NS
TPW = SEG // NW
CHUNK = 128
NCH = TPW // CHUNK
NG = CHUNK // 16
ACC = NS * N_EXPERTS

NCHAIN = 4
EPC = N_EXPERTS // NCHAIN


def _mlp_kernel(xa_ref, xb_ref, W1a_ref, W1b_ref, b1_ref, g_ref, be_ref,
                W2_ref, b2_ref, out_ref):
    h = jnp.dot(xa_ref[...], W1a_ref[...],
                preferred_element_type=jnp.float32)
    h = h + jnp.dot(xb_ref[...], W1b_ref[...],
                    preferred_element_type=jnp.float32)
    h = h + b1_ref[...]
    mu = jnp.mean(h, axis=-1, keepdims=True)
    c = h - mu
    var = jnp.mean(c * c, axis=-1, keepdims=True)
    hn = c / jnp.sqrt(var + 1e-5) * g_ref[...] + be_ref[...]
    hg = 0.5 * hn * (1.0 + jax.lax.erf(hn * 0.7071067811865476))
    logits_t = lax.dot_general(W2_ref[...], hg, (((0,), (1,)), ((), ())),
                               preferred_element_type=jnp.float32)
    out_ref[...] = logits_t + b2_ref[...]


def _merge2(a, b):
    am1, ai1, am2, ai2 = a
    bm1, bi1, bm2, bi2 = b
    c = bm1 > am1
    m1 = jnp.where(c, bm1, am1)
    i1 = jnp.where(c, bi1, ai1)
    cand = jnp.where(c, am1, bm1)
    candi = jnp.where(c, ai1, bi1)
    om2 = jnp.where(c, bm2, am2)
    omi = jnp.where(c, bi2, ai2)
    c2 = cand >= om2
    m2 = jnp.where(c2, cand, om2)
    i2 = jnp.where(c2, candi, omi)
    return (m1, i1, m2, i2)


def _route_kernel(lg_hbm, w_hbm, f_hbm, p_hbm, lbuf, wbuf, facc, pacc,
                  lsem, wsem):
    wid = lax.axis_index("s") * NC + lax.axis_index("c")
    lane = lax.iota(jnp.int32, 16)
    lane64 = lane * N_EXPERTS
    zeros16 = jnp.zeros((16,), jnp.float32)
    ones16 = jnp.ones((16,), jnp.float32)

    def zacc(k, _):
        facc[pl.ds(k * 16, 16)] = zeros16
        pacc[pl.ds(k * 16, 16)] = zeros16
        return 0

    lax.fori_loop(0, ACC // 16, zacc, 0)

    base_tok = wid * TPW

    def start_in(c):
        return pltpu.async_copy(
            lg_hbm.at[:, pl.ds(base_tok + c * CHUNK, CHUNK)],
            lbuf.at[c % 2], lsem)

    in_cp = {0: start_in(0)}
    out_cp = {}
    for c in range(NCH):
        if c + 1 < NCH:
            in_cp[c + 1] = start_in(c + 1)
        if c >= 2:
            out_cp[c - 2].wait()
        lb = lbuf.at[c % 2]
        wb = wbuf.at[c % 2]

        def zw(k, _):
            for j in range(4):
                wb[k, pl.ds(j * 16, 16)] = zeros16
            return 0

        lax.fori_loop(0, CHUNK, zw, 0)
        in_cp[c].wait()

        def group(g, _):
            rows = lane + g * 16
            states = []
            for ch in range(NCHAIN):
                states.append((jnp.full((16,), -jnp.inf, jnp.float32),
                               jnp.zeros((16,), jnp.int32),
                               jnp.full((16,), -jnp.inf, jnp.float32),
                               jnp.zeros((16,), jnp.int32)))
            for e in range(EPC):
                for ch in range(NCHAIN):
                    ee = ch * EPC + e
                    v = lb[ee, pl.ds(g * 16, 16)]
                    m1, i1, m2, i2 = states[ch]
                    ev = jnp.full((16,), ee, jnp.int32)
                    c1 = v > m1
                    c2 = v > m2
                    m2 = jnp.where(c1, m1, jnp.where(c2, v, m2))
                    i2 = jnp.where(c1, i1, jnp.where(c2, ev, i2))
                    m1 = jnp.where(c1, v, m1)
                    i1 = jnp.where(c1, ev, i1)
                    states[ch] = (m1, i1, m2, i2)
            m1, i1, m2, i2 = _merge2(_merge2(states[0], states[1]),
                                     _merge2(states[2], states[3]))
            e2 = jnp.exp(m2 - m1)
            w1 = 1.0 / (1.0 + e2)
            w2 = e2 * w1
            plsc.store_scatter(wb, [rows, i1], w1)
            plsc.store_scatter(wb, [rows, i2], w2)
            f2 = (w2 > 0).astype(jnp.float32)
            plsc.addupdate_scatter(facc, [lane64 + i1], ones16)
            plsc.addupdate_scatter(facc, [lane64 + i2], f2)
            plsc.addupdate_scatter(pacc, [lane64 + i1], w1)
            plsc.addupdate_scatter(pacc, [lane64 + i2], w2)
            return 0

        lax.fori_loop(0, NG, group, 0)
        out_cp[c] = pltpu.async_copy(
            wb, w_hbm.at[pl.ds(base_tok + c * CHUNK, CHUNK)], wsem)

    out_cp[NCH - 2].wait()
    out_cp[NCH - 1].wait()
    pltpu.sync_copy(facc, f_hbm.at[wid])
    pltpu.sync_copy(pacc, p_hbm.at[wid])


def _lb_kernel(f_ref, p_ref, lb_ref):
    f = jnp.sum(f_ref[...], axis=0, keepdims=True)
    p = jnp.sum(p_ref[...], axis=0, keepdims=True)
    s = jnp.sum(f * p)
    lb_ref[...] = (LB_WEIGHT * N_EXPERTS / (TOKENS * TOKENS) * s
                   ).reshape(1, 1)


def _make_mlp(seg_idx):
    nblk = SEG // BT
    return pl.pallas_call(
        _mlp_kernel,
        grid=(nblk,),
        in_specs=[
            pl.BlockSpec((BT, D_IN // 2),
                         lambda i, s=seg_idx, n=nblk: (s * n + i, 0)),
            pl.BlockSpec((BT, D_IN // 2),
                         lambda i, s=seg_idx, n=nblk: (s * n + i, 1)),
            pl.BlockSpec((D_IN // 2, D_HID), lambda i: (0, 0)),
            pl.BlockSpec((D_IN // 2, D_HID), lambda i: (1, 0)),
            pl.BlockSpec((1, D_HID), lambda i: (0, 0)),
            pl.BlockSpec((1, D_HID), lambda i: (0, 0)),
            pl.BlockSpec((1, D_HID), lambda i: (0, 0)),
            pl.BlockSpec((D_HID, N_EXPERTS), lambda i: (0, 0)),
            pl.BlockSpec((N_EXPERTS, 1), lambda i: (0, 0)),
        ],
        out_specs=pl.BlockSpec((N_EXPERTS, BT), lambda i: (0, i)),
        out_shape=jax.ShapeDtypeStruct((N_EXPERTS, SEG), jnp.float32),
        compiler_params=pltpu.CompilerParams(
            dimension_semantics=("arbitrary",),
        ),
    )


_route = pl.kernel(
    _route_kernel,
    out_type=[
        jax.ShapeDtypeStruct((SEG, N_EXPERTS), jnp.float32),
        jax.ShapeDtypeStruct((NW, ACC), jnp.float32),
        jax.ShapeDtypeStruct((NW, ACC), jnp.float32),
    ],
    mesh=plsc.VectorSubcoreMesh(core_axis_name="c", subcore_axis_name="s"),
    scratch_types=[
        pltpu.VMEM((2, N_EXPERTS, CHUNK), jnp.float32),
        pltpu.VMEM((2, CHUNK, N_EXPERTS), jnp.float32),
        pltpu.VMEM((ACC,), jnp.float32),
        pltpu.VMEM((ACC,), jnp.float32),
        pltpu.SemaphoreType.DMA,
        pltpu.SemaphoreType.DMA,
    ],
    compiler_params=pltpu.CompilerParams(needs_layout_passes=False),
)


@functools.partial(jax.jit)
def kernel(x, W1, b1, gamma, beta, W2, b2):
    b1r = b1.reshape(1, D_HID)
    gr = gamma.reshape(1, D_HID)
    ber = beta.reshape(1, D_HID)
    b2r = b2.reshape(N_EXPERTS, 1)

    w_segs, f_segs, p_segs = [], [], []
    for s in range(NSPLIT):
        logits = _make_mlp(s)(x, x, W1, W1, b1r, gr, ber, W2, b2r)
        w_s, f_s, p_s = _route(logits)
        w_segs.append(w_s)
        f_segs.append(f_s)
        p_segs.append(p_s)

    weights = jnp.concatenate(w_segs, axis=0)
    fpart = jnp.concatenate(f_segs, axis=0).reshape(NSPLIT * NW * NS,
                                                    N_EXPERTS)
    ppart = jnp.concatenate(p_segs, axis=0).reshape(NSPLIT * NW * NS,
                                                    N_EXPERTS)

    lb = pl.pallas_call(
        _lb_kernel,
        in_specs=[
            pl.BlockSpec((NSPLIT * NW * NS, N_EXPERTS), lambda: (0, 0)),
            pl.BlockSpec((NSPLIT * NW * NS, N_EXPERTS), lambda: (0, 0)),
        ],
        out_specs=pl.BlockSpec((1, 1), lambda: (0, 0)),
        out_shape=jax.ShapeDtypeStruct((1, 1), jnp.float32),
    )(fpart, ppart)
    return weights, lb.reshape(())
